# baseline (speedup 1.0000x reference)
import jax
import jax.numpy as jnp
from jax import lax
from jax.experimental import pallas as pl
from jax.experimental.pallas import tpu as pltpu

BLK = 128
ROWS = 84


def _body(x_ref, p_ref, v_ref, e_min, o_max):
    i = pl.program_id(0)
    n = pl.num_programs(0)
    step = x_ref[:, 0, 0:1]
    v_ref[:, :] = 100.0 - step
    par = jnp.bitwise_and(step.astype(jnp.int32), 1)
    bo = jnp.max(par)
    be = jnp.min(par)

    @pl.when(i == 0)
    def _init():
        e_min[0] = be
        o_max[0] = bo

    @pl.when(i > 0)
    def _acc():
        e_min[0] = jnp.minimum(e_min[0], be)
        o_max[0] = jnp.maximum(o_max[0], bo)

    @pl.when(i == n - 1)
    def _fin():
        c0 = jnp.where(e_min[0] == 0, 0.8, 0.2)
        c1 = jnp.where(o_max[0] == 1, 0.8, 0.2)
        col = lax.broadcasted_iota(jnp.int32, (p_ref.shape[0], 2), 1)
        p_ref[:, :] = jnp.where(col == 0, c0, c1)


def kernel(x):
    B, C, H, W = x.shape
    P, v = pl.pallas_call(
        _body,
        grid=(B // BLK,),
        in_specs=[pl.BlockSpec((BLK, None, ROWS, W), lambda i: (i, 0, 0, 0))],
        out_specs=(
            pl.BlockSpec((B, 2), lambda i: (0, 0)),
            pl.BlockSpec((BLK, 1), lambda i: (i, 0)),
        ),
        out_shape=(
            jax.ShapeDtypeStruct((B, 2), jnp.float32),
            jax.ShapeDtypeStruct((B, 1), jnp.float32),
        ),
        scratch_shapes=[
            pltpu.SMEM((1,), jnp.int32),
            pltpu.SMEM((1,), jnp.int32),
        ],
    )(x)
    return (P, v)


# speedup vs baseline: 1.0519x; 1.0519x over previous
import jax
import jax.numpy as jnp
from jax import lax
from jax.experimental import pallas as pl
from jax.experimental.pallas import tpu as pltpu

BLK = 128


def _body(xp_ref, x_any, p_ref, v_ref, face, e_min, o_max, sem):
    i = pl.program_id(0)
    n = pl.num_programs(0)
    B = v_ref.shape[0]
    half = B // 2
    cp = pltpu.make_async_copy(
        x_any.at[pl.ds(half + i * BLK, BLK), 0, 0], face, sem
    )
    cp.start()
    s1 = xp_ref[:, 0, 0:1]
    v_ref[pl.ds(i * BLK, BLK), :] = 100.0 - s1
    p1 = jnp.bitwise_and(s1.astype(jnp.int32), 1)
    cp.wait()
    s2 = face[:, 0:1]
    v_ref[pl.ds(half + i * BLK, BLK), :] = 100.0 - s2
    p2 = jnp.bitwise_and(s2.astype(jnp.int32), 1)
    bo = jnp.maximum(jnp.max(p1), jnp.max(p2))
    be = jnp.minimum(jnp.min(p1), jnp.min(p2))

    @pl.when(i == 0)
    def _init():
        e_min[0] = be
        o_max[0] = bo

    @pl.when(i > 0)
    def _acc():
        e_min[0] = jnp.minimum(e_min[0], be)
        o_max[0] = jnp.maximum(o_max[0], bo)

    @pl.when(i == n - 1)
    def _fin():
        c0 = jnp.where(e_min[0] == 0, 0.8, 0.2)
        c1 = jnp.where(o_max[0] == 1, 0.8, 0.2)
        col = lax.broadcasted_iota(jnp.int32, (p_ref.shape[0], 2), 1)
        p_ref[:, :] = jnp.where(col == 0, c0, c1)


def kernel(x):
    B, C, H, W = x.shape
    P, v = pl.pallas_call(
        _body,
        grid=(B // 2 // BLK,),
        in_specs=[
            pl.BlockSpec((BLK, None, 8, W), lambda i: (i, 0, 0, 0)),
            pl.BlockSpec(memory_space=pl.ANY),
        ],
        out_specs=(
            pl.BlockSpec((B, 2), lambda i: (0, 0)),
            pl.BlockSpec((B, 1), lambda i: (0, 0)),
        ),
        out_shape=(
            jax.ShapeDtypeStruct((B, 2), jnp.float32),
            jax.ShapeDtypeStruct((B, 1), jnp.float32),
        ),
        scratch_shapes=[
            pltpu.VMEM((BLK, W), jnp.float32),
            pltpu.SMEM((1,), jnp.int32),
            pltpu.SMEM((1,), jnp.int32),
            pltpu.SemaphoreType.DMA,
        ],
    )(x, x)
    return (P, v)


# speedup vs baseline: 1.0825x; 1.0291x over previous
import jax
import jax.numpy as jnp
from jax import lax
from jax.experimental import pallas as pl
from jax.experimental.pallas import tpu as pltpu

NSTREAM = 16


def _body(x_hbm, p_ref, v_ref, *scratch):
    faces = scratch[:NSTREAM]
    sems = scratch[NSTREAM:]
    B = v_ref.shape[0]
    chunk = B // NSTREAM
    copies = []
    for k in range(NSTREAM):
        cp = pltpu.make_async_copy(
            x_hbm.at[pl.ds(k * chunk, chunk), 0, 0],
            faces[k],
            sems[k],
        )
        cp.start()
        copies.append(cp)
    for cp in copies:
        cp.wait()
    any_even = False
    any_odd = False
    for k in range(NSTREAM):
        step_k = faces[k][:, 0:1]
        v_ref[pl.ds(k * chunk, chunk), :] = 100.0 - step_k
        par_k = jnp.bitwise_and(step_k.astype(jnp.int32), 1)
        any_odd = jnp.logical_or(any_odd, jnp.max(par_k) > 0)
        any_even = jnp.logical_or(any_even, jnp.min(par_k) < 1)
    c0 = jnp.where(any_even, 0.8, 0.2)
    c1 = jnp.where(any_odd, 0.8, 0.2)
    col = lax.broadcasted_iota(jnp.int32, (B, 2), 1)
    p_ref[:, :] = jnp.where(col == 0, c0, c1)


def kernel(x):
    B = x.shape[0]
    W = x.shape[3]
    chunk = B // NSTREAM
    P, v = pl.pallas_call(
        _body,
        in_specs=[pl.BlockSpec(memory_space=pl.ANY)],
        out_specs=(
            pl.BlockSpec((B, 2), lambda: (0, 0)),
            pl.BlockSpec((B, 1), lambda: (0, 0)),
        ),
        out_shape=(
            jax.ShapeDtypeStruct((B, 2), jnp.float32),
            jax.ShapeDtypeStruct((B, 1), jnp.float32),
        ),
        scratch_shapes=(
            [pltpu.VMEM((chunk, W), jnp.float32) for _ in range(NSTREAM)]
            + [pltpu.SemaphoreType.DMA for _ in range(NSTREAM)]
        ),
    )(x)
    return (P, v)
